# baseline (speedup 1.0000x reference)
import functools

import jax
import jax.numpy as jnp
from jax import lax
from jax.experimental import pallas as pl
from jax.experimental.pallas import tpu as pltpu
from jax.experimental.pallas import tpu_sc as plsc

U = 25000
NI = 25000
N = U + NI
D = 64
H = 32
NC = 2
NS = 16
NPAD = 51200
TN = NPAD // NS
NBK = TN // 128
UPAD = 25600
NLAYERS = 3


def _proj_tc(Gi, F):
    def body(g_ref, f_ref, o_ref):
        o_ref[...] = jnp.dot(jax.nn.sigmoid(g_ref[...]), f_ref[...],
                             preferred_element_type=jnp.float32)
    rows = Gi.shape[0]
    blk = 1000
    return pl.pallas_call(
        body,
        grid=(rows // blk,),
        in_specs=[pl.BlockSpec((blk, D), lambda i: (i, 0)),
                  pl.BlockSpec((D, D), lambda i: (0, 0))],
        out_specs=pl.BlockSpec((blk, D), lambda i: (i, 0)),
        out_shape=jax.ShapeDtypeStruct((rows, D), jnp.float32),
    )(Gi, F)


def _make_sc(teb, b):
    bpt = b // NS
    bblk = bpt // 128
    mesh = plsc.VectorSubcoreMesh(core_axis_name="c", subcore_axis_name="s")

    @functools.partial(
        pl.kernel,
        out_type=[
            jax.ShapeDtypeStruct((NC * b,), jnp.float32),
            jax.ShapeDtypeStruct((NC * NPAD, H), jnp.float32),
            jax.ShapeDtypeStruct((NC * NPAD, H), jnp.float32),
            jax.ShapeDtypeStruct((NC * UPAD, H), jnp.float32),
        ],
        mesh=mesh,
        compiler_params=pltpu.CompilerParams(
            needs_layout_passes=False, use_tc_tiling_on_sc=False),
        scratch_types=[
            pltpu.VMEM_SHARED((NPAD, H), jnp.float32),
            pltpu.VMEM_SHARED((NPAD,), jnp.float32),
            pltpu.VMEM((1024,), jnp.int32),
            pltpu.VMEM((8, 128), jnp.int32),
            pltpu.VMEM((256, H), jnp.float32),
            pltpu.VMEM((256, H), jnp.float32),
            pltpu.VMEM((32, H), jnp.float32),
            pltpu.VMEM((TN,), jnp.float32),
            pltpu.VMEM((8, 128), jnp.float32),
            pltpu.VMEM((32, H), jnp.float32),
            pltpu.VMEM((128,), jnp.float32),
            pltpu.SemaphoreType.DMA,
            pltpu.SemaphoreType.DMA,
            pltpu.SemaphoreType.DMA,
            pltpu.SemaphoreType.DMA,
            pltpu.SemaphoreType.DMA,
            pltpu.SemaphoreType.DMA,
        ],
    )
    def sc(ego, row_r, col_r, projf, uidx2, iidx2,
           part, ya, yb, accu,
           acc_sp, deg_sp, ridx, cidx, g0, g1, nbuf,
           dinvb, ones2, zbuf, xb,
           gsem0, gsem1, gsem2, ssem0, ssem1, ssem2):
        c = lax.axis_index("c")
        s = lax.axis_index("s")
        nb0 = s * TN
        eb0 = s * teb

        vone = jnp.ones((16,), jnp.float32)
        vzero = jnp.zeros((16,), jnp.float32)

        def onerow(i, _):
            for h in range(8):
                ones2[i, pl.ds(h * 16, 16)] = vone
            return 0
        lax.fori_loop(0, 8, onerow, 0)

        def zrow(i, _):
            for h in range(H // 16):
                zbuf[i, pl.ds(h * 16, 16)] = vzero
            return 0
        lax.fori_loop(0, 32, zrow, 0)
        for g in range(8):
            xb[pl.ds(g * 16, 16)] = vzero

        def _zero_acc32(g):
            pltpu.sync_copy(zbuf, acc_sp.at[pl.ds(g, 32)])

        def zblk(i, _):
            g = nb0 + i * 32
            _zero_acc32(g)
            return 0
        lax.fori_loop(0, TN // 32, zblk, 0)

        def zdeg(i, _):
            pltpu.sync_copy(xb, deg_sp.at[pl.ds(nb0 + i * 128, 128)])
            return 0
        lax.fori_loop(0, TN // 128, zdeg, 0)
        plsc.subcore_barrier()

        def deg_outer(o, _):
            pltpu.sync_copy(col_r.at[pl.ds(eb0 + o * 8, 8)], cidx)
            descs = [pltpu.async_copy(ones2.at[j], deg_sp.at[cidx.at[j]],
                                      (ssem0, ssem1)[j % 2], add=True)
                     for j in range(8)]
            for d in descs:
                d.wait()
            return 0
        lax.fori_loop(0, teb // 8, deg_outer, 0)
        plsc.subcore_barrier()

        pltpu.sync_copy(deg_sp.at[pl.ds(nb0, TN)], dinvb)

        def dv(i, _):
            d = dinvb[pl.ds(i * 16, 16)]
            xi = lax.bitcast_convert_type(d, jnp.int32)
            yi = jnp.int32(0x5F3759DF) - (xi >> 1)
            y = lax.bitcast_convert_type(yi, jnp.float32)
            for _ in range(3):
                y = y * (1.5 - 0.5 * d * y * y)
            dinvb[pl.ds(i * 16, 16)] = jnp.where(d > 0.0, y, 0.0)
            return 0
        lax.fori_loop(0, TN // 16, dv, 0)

        rows16 = lax.iota(jnp.int32, 16)
        lane0 = rows16 == 0

        def _dinv_row(i):
            iv = jnp.zeros((16,), jnp.int32) + i
            return plsc.load_gather(dinvb, [iv])

        nstr = N % 32

        def _scale_store(bk, g):
            def rowb(i, _):
                dv16 = _dinv_row(bk * 32 + i)
                for h in range(H // 16):
                    v = nbuf[i, pl.ds(h * 16, 16)]
                    nbuf[i, pl.ds(h * 16, 16)] = v * dv16
                return 0
            lax.fori_loop(0, 32, rowb, 0)
            pltpu.sync_copy(nbuf, ya.at[pl.ds(c * NPAD + g, 32)])

        def init_b(bk, _):
            g = nb0 + bk * 32

            @pl.when(g + 32 <= N)
            def _():
                pltpu.sync_copy(ego.at[pl.ds(g, 32), pl.ds(c * H, H)], nbuf)

                @pl.when(s < 8)
                def _():
                    pltpu.sync_copy(nbuf, accu.at[pl.ds(c * UPAD + g, 32)])
                _scale_store(bk, g)

            @pl.when(g + 32 > N)
            def _():
                @pl.when(g < N)
                def _():
                    pltpu.sync_copy(ego.at[pl.ds(g, nstr), pl.ds(c * H, H)],
                                    nbuf.at[pl.ds(0, nstr)])

                def zr(i, _):
                    for h in range(H // 16):
                        nbuf[i, pl.ds(h * 16, 16)] = vzero
                    return 0
                lax.fori_loop(jnp.maximum(N - g, 0), 32, zr, 0)
                _scale_store(bk, g)
            return 0
        lax.fori_loop(0, TN // 32, init_b, 0)
        plsc.subcore_barrier()

        for k in range(NLAYERS):
            ycur, ynext = (ya, yb) if k % 2 == 0 else (yb, ya)
            alpha = 1.0 / (k + 2)
            last = k == NLAYERS - 1

            def edge_outer(o, _):
                base = eb0 + o * 8
                pltpu.sync_copy(row_r.at[pl.ds(base * 128, 1024)], ridx)
                pltpu.sync_copy(col_r.at[pl.ds(base, 8)], cidx)

                @pl.when(c > 0)
                def _():
                    coff = jnp.zeros((16,), jnp.int32) + c * NPAD

                    def addoff(r, _):
                        sl = ridx[pl.ds(r * 16, 16)]
                        ridx[pl.ds(r * 16, 16)] = sl + coff
                        return 0
                    lax.fori_loop(0, 64, addoff, 0)
                bufs = (g0, g1)
                gs = (gsem0, gsem1)
                ss = (ssem0, ssem1)
                gd = [None] * 4
                sd = [None] * 4
                gd[0] = pltpu.async_copy(ycur.at[ridx.at[pl.ds(0, 256)]],
                                         g0, gsem0)
                for q in range(4):
                    bq = q % 2
                    if q + 1 < 4:
                        nb_ = (q + 1) % 2
                        if q >= 1:
                            sd[q - 1][1].wait()
                        gd[q + 1] = pltpu.async_copy(
                            ycur.at[ridx.at[pl.ds(256 * (q + 1), 256)]],
                            bufs[nb_], gs[nb_])
                    gd[q].wait()
                    buf = bufs[bq]
                    sd[q] = (
                        pltpu.async_copy(buf.at[pl.ds(0, 128)],
                                         acc_sp.at[cidx.at[2 * q]],
                                         ss[bq], add=True),
                        pltpu.async_copy(buf.at[pl.ds(128, 128)],
                                         acc_sp.at[cidx.at[2 * q + 1]],
                                         ss[bq], add=True))
                sd[2][1].wait()
                sd[3][1].wait()
                return 0
            lax.fori_loop(0, teb // 8, edge_outer, 0)
            plsc.subcore_barrier()

            def node_b(bk, _):
                g = nb0 + bk * 32
                pltpu.sync_copy(acc_sp.at[pl.ds(g, 32)], nbuf)

                @pl.when(s < 8)
                def _():
                    pltpu.sync_copy(accu.at[pl.ds(c * UPAD + g, 32)],
                                    g0.at[pl.ds(0, 32)])

                    def rowa(i, _):
                        da = _dinv_row(bk * 32 + i) * alpha
                        for h in range(H // 16):
                            v = nbuf[i, pl.ds(h * 16, 16)]
                            g0[i, pl.ds(h * 16, 16)] = (
                                g0[i, pl.ds(h * 16, 16)] + v * da)
                        return 0
                    lax.fori_loop(0, 32, rowa, 0)
                    pltpu.sync_copy(g0.at[pl.ds(0, 32)],
                                    accu.at[pl.ds(c * UPAD + g, 32)])

                if not last:
                    _zero_acc32(g)

                    def rowb(i, _):
                        dv16 = _dinv_row(bk * 32 + i)
                        z = dv16 * dv16
                        for h in range(H // 16):
                            v = nbuf[i, pl.ds(h * 16, 16)]
                            nbuf[i, pl.ds(h * 16, 16)] = v * z
                        return 0
                    lax.fori_loop(0, 32, rowb, 0)
                    pltpu.sync_copy(nbuf, ynext.at[pl.ds(c * NPAD + g, 32)])
                return 0
            lax.fori_loop(0, TN // 32, node_b, 0)
            plsc.subcore_barrier()

        pltpu.sync_copy(uidx2.at[c].at[pl.ds(s * bpt, bpt)], ridx)
        pltpu.sync_copy(iidx2.at[c].at[pl.ds(s * bblk, bblk)], cidx)
        for bb in range(bblk):
            du = pltpu.async_copy(accu.at[ridx.at[pl.ds(bb * 128, 128)]],
                                  g0.at[pl.ds(0, 128)], gsem0)
            di = pltpu.async_copy(projf.at[cidx.at[bb]],
                                  g1.at[pl.ds(0, 128)], gsem1)
            du.wait()
            di.wait()

            def rowd(i, _):
                v = jnp.zeros((16,), jnp.float32)
                for h in range(H // 16):
                    v = v + (g0[i, pl.ds(h * 16, 16)] *
                             g1[i, pl.ds(h * 16, 16)])
                tot = jnp.sum(v)
                iv = jnp.zeros((16,), jnp.int32) + i
                plsc.store_scatter(xb, [iv],
                                   jnp.zeros((16,), jnp.float32) + tot,
                                   mask=lane0)
                return 0
            lax.fori_loop(0, 128, rowd, 0)
            pltpu.sync_copy(
                xb, part.at[pl.ds(c * b + s * bpt + bb * 128, 128)])

    return sc


def kernel(Gu, Gi, F, edge_index, user_idx, item_idx):
    proj = _proj_tc(Gi, F)
    ego = jnp.concatenate([Gu, proj], axis=0)

    e = edge_index.shape[1]
    chunk = 128 * NS * 8
    epad = ((e + chunk - 1) // chunk) * chunk
    p = epad - e
    pidx = jnp.arange(p, dtype=jnp.int32)
    row_r = jnp.concatenate([edge_index[0], pidx % N])
    col_r = jnp.concatenate([edge_index[1],
                             N + (pidx % 512)]).reshape(epad // 128, 128)

    b = user_idx.shape[0]
    uidx2 = jnp.stack([user_idx, user_idx + UPAD])
    iidx2 = jnp.stack([item_idx, item_idx + NI]).reshape(NC, b // 128, 128)
    projf = jnp.concatenate([proj[:, :H], proj[:, H:]], axis=0)

    sc = _make_sc((epad // 128) // NS, b)
    part, _, _, _ = sc(ego, row_r, col_r, projf, uidx2, iidx2)
    return part[:b] + part[b:]


# speedup vs baseline: 1.0456x; 1.0456x over previous
import functools

import jax
import jax.numpy as jnp
from jax import lax
from jax.experimental import pallas as pl
from jax.experimental.pallas import tpu as pltpu
from jax.experimental.pallas import tpu_sc as plsc

U = 25000
NI = 25000
N = U + NI
D = 64
H = 32
NC = 2
NS = 16
NPAD = 51200
TN = NPAD // NS
NBK = TN // 128
UPAD = 25600
NLAYERS = 3


def _proj_tc(Gi, F):
    def body(g_ref, f_ref, o_ref):
        o_ref[...] = jnp.dot(jax.nn.sigmoid(g_ref[...]), f_ref[...],
                             preferred_element_type=jnp.float32)
    rows = Gi.shape[0]
    blk = 1000
    return pl.pallas_call(
        body,
        grid=(rows // blk,),
        in_specs=[pl.BlockSpec((blk, D), lambda i: (i, 0)),
                  pl.BlockSpec((D, D), lambda i: (0, 0))],
        out_specs=pl.BlockSpec((blk, D), lambda i: (i, 0)),
        out_shape=jax.ShapeDtypeStruct((rows, D), jnp.float32),
    )(Gi, F)


def _make_sc(teb, b):
    bpt = b // NS
    bblk = bpt // 128
    mesh = plsc.VectorSubcoreMesh(core_axis_name="c", subcore_axis_name="s")

    @functools.partial(
        pl.kernel,
        out_type=[
            jax.ShapeDtypeStruct((NC * b,), jnp.float32),
            jax.ShapeDtypeStruct((NC * NPAD, H), jnp.float32),
            jax.ShapeDtypeStruct((NC * NPAD, H), jnp.float32),
            jax.ShapeDtypeStruct((NC * UPAD, H), jnp.float32),
        ],
        mesh=mesh,
        compiler_params=pltpu.CompilerParams(
            needs_layout_passes=False, use_tc_tiling_on_sc=False),
        scratch_types=[
            pltpu.VMEM_SHARED((NPAD, H), jnp.float32),
            pltpu.VMEM_SHARED((NPAD,), jnp.float32),
            pltpu.VMEM((8, 128), jnp.int32),
            pltpu.VMEM((8, 128), jnp.int32),
            pltpu.VMEM((128, H), jnp.float32),
            pltpu.VMEM((128, H), jnp.float32),
            pltpu.VMEM((128, H), jnp.float32),
            pltpu.VMEM((128, H), jnp.float32),
            pltpu.VMEM((64, H), jnp.float32),
            pltpu.VMEM((TN,), jnp.float32),
            pltpu.VMEM((128,), jnp.float32),
            pltpu.VMEM((128,), jnp.float32),
            pltpu.VMEM((32, H), jnp.float32),
            pltpu.VMEM((128,), jnp.float32),
            pltpu.SemaphoreType.DMA,
            pltpu.SemaphoreType.DMA,
            pltpu.SemaphoreType.DMA,
            pltpu.SemaphoreType.DMA,
            pltpu.SemaphoreType.DMA,
            pltpu.SemaphoreType.DMA,
        ],
    )
    def sc(ego, row_r, col_r, projf, uidx2, iidx2,
           part, ya, yb, accu,
           acc_sp, deg_sp, ridx, cidx, g0, g1, g2, g3, nbuf,
           dinvb, onesb, z1, zbuf, xb,
           gsem0, gsem1, gsem2, gsem3, ssem0, ssem1):
        c = lax.axis_index("c")
        s = lax.axis_index("s")
        nb0 = s * TN
        eb0 = s * teb

        vone = jnp.ones((16,), jnp.float32)
        vzero = jnp.zeros((16,), jnp.float32)
        for g in range(8):
            onesb[pl.ds(g * 16, 16)] = vone
            z1[pl.ds(g * 16, 16)] = vzero

        def zrow(i, _):
            for h in range(H // 16):
                zbuf[i, pl.ds(h * 16, 16)] = vzero
            return 0
        lax.fori_loop(0, 32, zrow, 0)

        def _zero_acc(g):
            pltpu.sync_copy(zbuf, acc_sp.at[pl.ds(g, 32)])
            pltpu.sync_copy(zbuf, acc_sp.at[pl.ds(g + 32, 32)])

        def zblk(i, _):
            g = nb0 + i * 64
            pltpu.sync_copy(zbuf, acc_sp.at[pl.ds(g, 32)])
            pltpu.sync_copy(zbuf, acc_sp.at[pl.ds(g + 32, 32)])
            return 0
        lax.fori_loop(0, TN // 64, zblk, 0)

        def zdeg(i, _):
            pltpu.sync_copy(z1, deg_sp.at[pl.ds(nb0 + i * 128, 128)])
            return 0
        lax.fori_loop(0, TN // 128, zdeg, 0)
        plsc.subcore_barrier()

        def deg_outer(o, _):
            pltpu.sync_copy(col_r.at[pl.ds(eb0 + o * 8, 8)], cidx)
            descs = [pltpu.async_copy(onesb, deg_sp.at[cidx.at[j]],
                                      (ssem0, ssem1)[j % 2], add=True)
                     for j in range(8)]
            for d in descs:
                d.wait()
            return 0
        lax.fori_loop(0, teb // 8, deg_outer, 0)
        plsc.subcore_barrier()

        pltpu.sync_copy(deg_sp.at[pl.ds(nb0, TN)], dinvb)

        def dv(i, _):
            d = dinvb[pl.ds(i * 16, 16)]
            xi = lax.bitcast_convert_type(d, jnp.int32)
            yi = jnp.int32(0x5F3759DF) - (xi >> 1)
            y = lax.bitcast_convert_type(yi, jnp.float32)
            for _ in range(3):
                y = y * (1.5 - 0.5 * d * y * y)
            dinvb[pl.ds(i * 16, 16)] = jnp.where(d > 0.0, y, 0.0)
            return 0
        lax.fori_loop(0, TN // 16, dv, 0)

        rows16 = lax.iota(jnp.int32, 16)
        lane0 = rows16 == 0

        def _dinv_row(i):
            iv = jnp.zeros((16,), jnp.int32) + i
            return plsc.load_gather(dinvb, [iv])

        nstr = N % 64

        def _scale_store(bk, g):
            def rowb(i, _):
                dv16 = _dinv_row(bk * 64 + i)
                for h in range(H // 16):
                    v = nbuf[i, pl.ds(h * 16, 16)]
                    nbuf[i, pl.ds(h * 16, 16)] = v * dv16
                return 0
            lax.fori_loop(0, 64, rowb, 0)
            pltpu.sync_copy(nbuf, ya.at[pl.ds(c * NPAD + g, 64)])

        def init_b(bk, _):
            g = nb0 + bk * 64

            @pl.when(g + 64 <= N)
            def _():
                pltpu.sync_copy(ego.at[pl.ds(g, 64), pl.ds(c * H, H)], nbuf)

                @pl.when(s < 8)
                def _():
                    pltpu.sync_copy(nbuf, accu.at[pl.ds(c * UPAD + g, 64)])
                _scale_store(bk, g)

            @pl.when(g + 64 > N)
            def _():
                @pl.when(g < N)
                def _():
                    pltpu.sync_copy(ego.at[pl.ds(g, nstr), pl.ds(c * H, H)],
                                    nbuf.at[pl.ds(0, nstr)])

                def zr(i, _):
                    for h in range(H // 16):
                        nbuf[i, pl.ds(h * 16, 16)] = vzero
                    return 0
                lax.fori_loop(jnp.maximum(N - g, 0), 64, zr, 0)
                _scale_store(bk, g)
            return 0
        lax.fori_loop(0, TN // 64, init_b, 0)
        plsc.subcore_barrier()

        for k in range(NLAYERS):
            ycur, ynext = (ya, yb) if k % 2 == 0 else (yb, ya)
            alpha = 1.0 / (k + 2)
            last = k == NLAYERS - 1

            def edge_outer(o, _):
                base = eb0 + o * 8
                pltpu.sync_copy(row_r.at[pl.ds(base, 8)], ridx)
                pltpu.sync_copy(col_r.at[pl.ds(base, 8)], cidx)

                @pl.when(c > 0)
                def _():
                    coff = jnp.zeros((16,), jnp.int32) + c * NPAD

                    def addoff(r, _):
                        for gch in range(8):
                            sl = ridx[r, pl.ds(gch * 16, 16)]
                            ridx[r, pl.ds(gch * 16, 16)] = sl + coff
                        return 0
                    lax.fori_loop(0, 8, addoff, 0)
                bufs = (g0, g1, g2, g3)
                gs = (gsem0, gsem1, gsem2, gsem3)
                ss = (ssem0, ssem1)
                gd = [None] * 8
                sd = [None] * 8
                for j in range(3):
                    gd[j] = pltpu.async_copy(ycur.at[ridx.at[j]],
                                             bufs[j], gs[j])
                for j in range(8):
                    bj = j % 4
                    gd[j].wait()
                    sd[j] = pltpu.async_copy(bufs[bj],
                                             acc_sp.at[cidx.at[j]],
                                             ss[j % 2], add=True)
                    if j + 3 < 8:
                        nb_ = (j + 3) % 4
                        if j >= 1:
                            sd[j - 1].wait()
                        gd[j + 3] = pltpu.async_copy(
                            ycur.at[ridx.at[j + 3]], bufs[nb_], gs[nb_])
                sd[4].wait()
                sd[5].wait()
                sd[6].wait()
                sd[7].wait()
                return 0
            lax.fori_loop(0, teb // 8, edge_outer, 0)
            plsc.subcore_barrier()

            def node_b(bk, _):
                g = nb0 + bk * 64
                pltpu.sync_copy(acc_sp.at[pl.ds(g, 64)], nbuf)

                @pl.when(s < 8)
                def _():
                    pltpu.sync_copy(accu.at[pl.ds(c * UPAD + g, 64)],
                                    g0.at[pl.ds(0, 64)])

                    def rowa(i, _):
                        da = _dinv_row(bk * 64 + i) * alpha
                        for h in range(H // 16):
                            v = nbuf[i, pl.ds(h * 16, 16)]
                            g0[i, pl.ds(h * 16, 16)] = (
                                g0[i, pl.ds(h * 16, 16)] + v * da)
                        return 0
                    lax.fori_loop(0, 64, rowa, 0)
                    pltpu.sync_copy(g0.at[pl.ds(0, 64)],
                                    accu.at[pl.ds(c * UPAD + g, 64)])

                if not last:
                    _zero_acc(g)

                    def rowb(i, _):
                        dv16 = _dinv_row(bk * 64 + i)
                        z = dv16 * dv16
                        for h in range(H // 16):
                            v = nbuf[i, pl.ds(h * 16, 16)]
                            nbuf[i, pl.ds(h * 16, 16)] = v * z
                        return 0
                    lax.fori_loop(0, 64, rowb, 0)
                    pltpu.sync_copy(nbuf, ynext.at[pl.ds(c * NPAD + g, 64)])
                return 0
            lax.fori_loop(0, TN // 64, node_b, 0)
            plsc.subcore_barrier()

        pltpu.sync_copy(uidx2.at[c].at[pl.ds(s * bblk, bblk)], ridx)
        pltpu.sync_copy(iidx2.at[c].at[pl.ds(s * bblk, bblk)], cidx)
        for bb in range(bblk):
            du = pltpu.async_copy(accu.at[ridx.at[bb]], g0, gsem0)
            di = pltpu.async_copy(projf.at[cidx.at[bb]], g1, gsem1)
            du.wait()
            di.wait()

            def rowd(i, _):
                v = jnp.zeros((16,), jnp.float32)
                for h in range(H // 16):
                    v = v + (g0[i, pl.ds(h * 16, 16)] *
                             g1[i, pl.ds(h * 16, 16)])
                tot = jnp.sum(v)
                iv = jnp.zeros((16,), jnp.int32) + i
                plsc.store_scatter(xb, [iv],
                                   jnp.zeros((16,), jnp.float32) + tot,
                                   mask=lane0)
                return 0
            lax.fori_loop(0, 128, rowd, 0)
            pltpu.sync_copy(
                xb, part.at[pl.ds(c * b + s * bpt + bb * 128, 128)])

    return sc


def kernel(Gu, Gi, F, edge_index, user_idx, item_idx):
    proj = _proj_tc(Gi, F)
    ego = jnp.concatenate([Gu, proj], axis=0)

    e = edge_index.shape[1]
    chunk = 128 * NS * 8
    epad = ((e + chunk - 1) // chunk) * chunk
    p = epad - e
    pidx = jnp.arange(p, dtype=jnp.int32)
    row_r = jnp.concatenate([edge_index[0], pidx % N]).reshape(epad // 128,
                                                              128)
    col_r = jnp.concatenate([edge_index[1],
                             N + (pidx % 512)]).reshape(epad // 128, 128)

    b = user_idx.shape[0]
    uidx2 = jnp.stack([user_idx, user_idx + UPAD]).reshape(NC, b // 128, 128)
    iidx2 = jnp.stack([item_idx, item_idx + NI]).reshape(NC, b // 128, 128)
    projf = jnp.concatenate([proj[:, :H], proj[:, H:]], axis=0)

    sc = _make_sc((epad // 128) // NS, b)
    part, _, _, _ = sc(ego, row_r, col_r, projf, uidx2, iidx2)
    return part[:b] + part[b:]


# speedup vs baseline: 1.0606x; 1.0143x over previous
import functools

import jax
import jax.numpy as jnp
from jax import lax
from jax.experimental import pallas as pl
from jax.experimental.pallas import tpu as pltpu
from jax.experimental.pallas import tpu_sc as plsc

U = 25000
NI = 25000
N = U + NI
D = 64
H = 32
NC = 2
NS = 16
NPAD = 51200
TN = NPAD // NS
NBK = TN // 128
UPAD = 25600
NLAYERS = 3


def _proj_tc(Gi, F):
    def body(g_ref, f_ref, o_ref):
        o_ref[...] = jnp.dot(jax.nn.sigmoid(g_ref[...]), f_ref[...],
                             preferred_element_type=jnp.float32)
    rows = Gi.shape[0]
    blk = 1000
    return pl.pallas_call(
        body,
        grid=(rows // blk,),
        in_specs=[pl.BlockSpec((blk, D), lambda i: (i, 0)),
                  pl.BlockSpec((D, D), lambda i: (0, 0))],
        out_specs=pl.BlockSpec((blk, D), lambda i: (i, 0)),
        out_shape=jax.ShapeDtypeStruct((rows, D), jnp.float32),
    )(Gi, F)


def _make_sc(teb, b):
    bpt = b // NS
    bblk = bpt // 128
    mesh = plsc.VectorSubcoreMesh(core_axis_name="c", subcore_axis_name="s")

    @functools.partial(
        pl.kernel,
        out_type=[
            jax.ShapeDtypeStruct((NC * b,), jnp.float32),
            jax.ShapeDtypeStruct((NC * NPAD, H), jnp.float32),
            jax.ShapeDtypeStruct((NC * NPAD, H), jnp.float32),
            jax.ShapeDtypeStruct((NC * UPAD, H), jnp.float32),
        ],
        mesh=mesh,
        compiler_params=pltpu.CompilerParams(
            needs_layout_passes=False, use_tc_tiling_on_sc=False),
        scratch_types=[
            pltpu.VMEM_SHARED((NPAD, H), jnp.float32),
            pltpu.VMEM_SHARED((NPAD,), jnp.float32),
            pltpu.VMEM((8, 128), jnp.int32),
            pltpu.VMEM((8, 128), jnp.int32),
            pltpu.VMEM((128, H), jnp.float32),
            pltpu.VMEM((128, H), jnp.float32),
            pltpu.VMEM((128, H), jnp.float32),
            pltpu.VMEM((128, H), jnp.float32),
            pltpu.VMEM((64, H), jnp.float32),
            pltpu.VMEM((TN,), jnp.float32),
            pltpu.VMEM((128,), jnp.float32),
            pltpu.VMEM((128,), jnp.float32),
            pltpu.VMEM((32, H), jnp.float32),
            pltpu.VMEM((128,), jnp.float32),
            pltpu.SemaphoreType.DMA,
            pltpu.SemaphoreType.DMA,
            pltpu.SemaphoreType.DMA,
            pltpu.SemaphoreType.DMA,
            pltpu.SemaphoreType.DMA,
            pltpu.SemaphoreType.DMA,
        ],
    )
    def sc(gu, pj, row_r, col_r, projf, uidx2, iidx2,
           part, ya, yb, accu,
           acc_sp, deg_sp, ridx, cidx, g0, g1, g2, g3, nbuf,
           dinvb, onesb, z1, zbuf, xb,
           gsem0, gsem1, gsem2, gsem3, ssem0, ssem1):
        c = lax.axis_index("c")
        s = lax.axis_index("s")
        nb0 = s * TN
        eb0 = s * teb

        vone = jnp.ones((16,), jnp.float32)
        vzero = jnp.zeros((16,), jnp.float32)
        for g in range(8):
            onesb[pl.ds(g * 16, 16)] = vone
            z1[pl.ds(g * 16, 16)] = vzero

        def zrow(i, _):
            for h in range(H // 16):
                zbuf[i, pl.ds(h * 16, 16)] = vzero
            return 0
        lax.fori_loop(0, 32, zrow, 0)

        def _zero_acc(g):
            pltpu.sync_copy(zbuf, acc_sp.at[pl.ds(g, 32)])
            pltpu.sync_copy(zbuf, acc_sp.at[pl.ds(g + 32, 32)])

        def zblk(i, _):
            g = nb0 + i * 64
            pltpu.sync_copy(zbuf, acc_sp.at[pl.ds(g, 32)])
            pltpu.sync_copy(zbuf, acc_sp.at[pl.ds(g + 32, 32)])
            return 0
        lax.fori_loop(0, TN // 64, zblk, 0)

        def zdeg(i, _):
            pltpu.sync_copy(z1, deg_sp.at[pl.ds(nb0 + i * 128, 128)])
            return 0
        lax.fori_loop(0, TN // 128, zdeg, 0)
        plsc.subcore_barrier()

        def deg_outer(o, _):
            pltpu.sync_copy(col_r.at[pl.ds(eb0 + o * 8, 8)], cidx)
            descs = [pltpu.async_copy(onesb, deg_sp.at[cidx.at[j]],
                                      (ssem0, ssem1)[j % 2], add=True)
                     for j in range(8)]
            for d in descs:
                d.wait()
            return 0
        lax.fori_loop(0, teb // 8, deg_outer, 0)
        plsc.subcore_barrier()

        pltpu.sync_copy(deg_sp.at[pl.ds(nb0, TN)], dinvb)

        def dv(i, _):
            d = dinvb[pl.ds(i * 16, 16)]
            xi = lax.bitcast_convert_type(d, jnp.int32)
            yi = jnp.int32(0x5F3759DF) - (xi >> 1)
            y = lax.bitcast_convert_type(yi, jnp.float32)
            for _ in range(3):
                y = y * (1.5 - 0.5 * d * y * y)
            dinvb[pl.ds(i * 16, 16)] = jnp.where(d > 0.0, y, 0.0)
            return 0
        lax.fori_loop(0, TN // 16, dv, 0)

        rows16 = lax.iota(jnp.int32, 16)
        lane0 = rows16 == 0

        def _dinv_row(i):
            iv = jnp.zeros((16,), jnp.int32) + i
            return plsc.load_gather(dinvb, [iv])

        nstr = N % 64

        def _scale_store(bk, g):
            def rowb(i, _):
                dv16 = _dinv_row(bk * 64 + i)
                for h in range(H // 16):
                    v = nbuf[i, pl.ds(h * 16, 16)]
                    nbuf[i, pl.ds(h * 16, 16)] = v * dv16
                return 0
            lax.fori_loop(0, 64, rowb, 0)
            pltpu.sync_copy(nbuf, ya.at[pl.ds(c * NPAD + g, 64)])

        ustr = U % 64

        def init_b(bk, _):
            g = nb0 + bk * 64

            @pl.when(g + 64 <= U)
            def _():
                pltpu.sync_copy(gu.at[pl.ds(g, 64), pl.ds(c * H, H)], nbuf)

                @pl.when(s < 8)
                def _():
                    pltpu.sync_copy(nbuf, accu.at[pl.ds(c * UPAD + g, 64)])
                _scale_store(bk, g)

            @pl.when((g < U) & (g + 64 > U))
            def _():
                pltpu.sync_copy(gu.at[pl.ds(g, ustr), pl.ds(c * H, H)],
                                nbuf.at[pl.ds(0, ustr)])
                pltpu.sync_copy(pj.at[pl.ds(g + ustr - U, 64 - ustr),
                                      pl.ds(c * H, H)],
                                nbuf.at[pl.ds(ustr, 64 - ustr)])

                @pl.when(s < 8)
                def _():
                    pltpu.sync_copy(nbuf, accu.at[pl.ds(c * UPAD + g, 64)])
                _scale_store(bk, g)

            @pl.when((g >= U) & (g + 64 <= N))
            def _():
                pltpu.sync_copy(pj.at[pl.ds(g - U, 64), pl.ds(c * H, H)],
                                nbuf)

                @pl.when(s < 8)
                def _():
                    pltpu.sync_copy(nbuf, accu.at[pl.ds(c * UPAD + g, 64)])
                _scale_store(bk, g)

            @pl.when(g + 64 > N)
            def _():
                @pl.when(g < N)
                def _():
                    pltpu.sync_copy(pj.at[pl.ds(g - U, nstr),
                                          pl.ds(c * H, H)],
                                    nbuf.at[pl.ds(0, nstr)])

                def zr(i, _):
                    for h in range(H // 16):
                        nbuf[i, pl.ds(h * 16, 16)] = vzero
                    return 0
                lax.fori_loop(jnp.maximum(N - g, 0), 64, zr, 0)
                _scale_store(bk, g)
            return 0
        lax.fori_loop(0, TN // 64, init_b, 0)
        plsc.subcore_barrier()

        for k in range(NLAYERS):
            ycur, ynext = (ya, yb) if k % 2 == 0 else (yb, ya)
            alpha = 1.0 / (k + 2)
            last = k == NLAYERS - 1

            def edge_outer(o, _):
                base = eb0 + o * 8
                pltpu.sync_copy(row_r.at[pl.ds(base, 8)], ridx)
                pltpu.sync_copy(col_r.at[pl.ds(base, 8)], cidx)

                @pl.when(c > 0)
                def _():
                    coff = jnp.zeros((16,), jnp.int32) + c * NPAD

                    def addoff(r, _):
                        for gch in range(8):
                            sl = ridx[r, pl.ds(gch * 16, 16)]
                            ridx[r, pl.ds(gch * 16, 16)] = sl + coff
                        return 0
                    lax.fori_loop(0, 8, addoff, 0)
                bufs = (g0, g1, g2, g3)
                gs = (gsem0, gsem1, gsem2, gsem3)
                ss = (ssem0, ssem1)
                gd = [None] * 8
                sd = [None] * 8
                for j in range(3):
                    gd[j] = pltpu.async_copy(ycur.at[ridx.at[j]],
                                             bufs[j], gs[j])
                for j in range(8):
                    bj = j % 4
                    gd[j].wait()
                    sd[j] = pltpu.async_copy(bufs[bj],
                                             acc_sp.at[cidx.at[j]],
                                             ss[j % 2], add=True)
                    if j + 3 < 8:
                        nb_ = (j + 3) % 4
                        if j >= 1:
                            sd[j - 1].wait()
                        gd[j + 3] = pltpu.async_copy(
                            ycur.at[ridx.at[j + 3]], bufs[nb_], gs[nb_])
                sd[4].wait()
                sd[5].wait()
                sd[6].wait()
                sd[7].wait()
                return 0
            lax.fori_loop(0, teb // 8, edge_outer, 0)
            plsc.subcore_barrier()

            def node_b(bk, _):
                g = nb0 + bk * 64
                pltpu.sync_copy(acc_sp.at[pl.ds(g, 64)], nbuf)

                @pl.when(s < 8)
                def _():
                    pltpu.sync_copy(accu.at[pl.ds(c * UPAD + g, 64)],
                                    g0.at[pl.ds(0, 64)])

                    def rowa(i, _):
                        da = _dinv_row(bk * 64 + i) * alpha
                        for h in range(H // 16):
                            v = nbuf[i, pl.ds(h * 16, 16)]
                            g0[i, pl.ds(h * 16, 16)] = (
                                g0[i, pl.ds(h * 16, 16)] + v * da)
                        return 0
                    lax.fori_loop(0, 64, rowa, 0)
                    pltpu.sync_copy(g0.at[pl.ds(0, 64)],
                                    accu.at[pl.ds(c * UPAD + g, 64)])

                if not last:
                    _zero_acc(g)

                    def rowb(i, _):
                        dv16 = _dinv_row(bk * 64 + i)
                        z = dv16 * dv16
                        for h in range(H // 16):
                            v = nbuf[i, pl.ds(h * 16, 16)]
                            nbuf[i, pl.ds(h * 16, 16)] = v * z
                        return 0
                    lax.fori_loop(0, 64, rowb, 0)
                    pltpu.sync_copy(nbuf, ynext.at[pl.ds(c * NPAD + g, 64)])
                return 0
            lax.fori_loop(0, TN // 64, node_b, 0)
            plsc.subcore_barrier()

        pltpu.sync_copy(uidx2.at[c].at[pl.ds(s * bblk, bblk)], ridx)
        pltpu.sync_copy(iidx2.at[c].at[pl.ds(s * bblk, bblk)], cidx)
        for bb in range(bblk):
            du = pltpu.async_copy(accu.at[ridx.at[bb]], g0, gsem0)
            di = pltpu.async_copy(projf.at[cidx.at[bb]], g1, gsem1)
            du.wait()
            di.wait()

            def rowd(i, _):
                v = jnp.zeros((16,), jnp.float32)
                for h in range(H // 16):
                    v = v + (g0[i, pl.ds(h * 16, 16)] *
                             g1[i, pl.ds(h * 16, 16)])
                tot = jnp.sum(v)
                iv = jnp.zeros((16,), jnp.int32) + i
                plsc.store_scatter(xb, [iv],
                                   jnp.zeros((16,), jnp.float32) + tot,
                                   mask=lane0)
                return 0
            lax.fori_loop(0, 128, rowd, 0)
            pltpu.sync_copy(
                xb, part.at[pl.ds(c * b + s * bpt + bb * 128, 128)])

    return sc


def kernel(Gu, Gi, F, edge_index, user_idx, item_idx):
    proj = _proj_tc(Gi, F)

    e = edge_index.shape[1]
    chunk = 128 * NS * 8
    epad = ((e + chunk - 1) // chunk) * chunk
    p = epad - e
    pidx = jnp.arange(p, dtype=jnp.int32)
    row_r = jnp.concatenate([edge_index[0], pidx % N]).reshape(epad // 128,
                                                              128)
    col_r = jnp.concatenate([edge_index[1],
                             N + (pidx % 512)]).reshape(epad // 128, 128)

    b = user_idx.shape[0]
    uidx2 = jnp.stack([user_idx, user_idx + UPAD]).reshape(NC, b // 128, 128)
    iidx2 = jnp.stack([item_idx, item_idx + NI]).reshape(NC, b // 128, 128)
    projf = jnp.concatenate([proj[:, :H], proj[:, H:]], axis=0)

    sc = _make_sc((epad // 128) // NS, b)
    part, _, _, _ = sc(Gu, proj, row_r, col_r, projf, uidx2, iidx2)
    return part[:b] + part[b:]
